# Initial kernel scaffold; baseline (speedup 1.0000x reference)
import dataclasses
import functools

import jax
import jax.numpy as jnp
from jax import lax
from jax.experimental import pallas as pl
from jax.experimental.pallas import tpu as pltpu
from jax.experimental.pallas import tpu_sc as plsc

N = 10000
E = 320000
D = 128
NPAD = 10016
HALF = 5000
HALFPAD = 5008
NSUB = 16
CH = 2000
EDGES_PER_SLOT = E // NSUB
NCHUNK = EDGES_PER_SLOT // CH
NEG = -3e38
F32 = jnp.float32


def _sc_compiler_params():
    cp = pltpu.CompilerParams()
    if "needs_layout_passes" in pltpu.CompilerParams.__dataclass_fields__:
        cp = dataclasses.replace(cp, needs_layout_passes=False)
    return cp


def _make_sc_edge(d):
    mesh = plsc.VectorSubcoreMesh(core_axis_name="c", subcore_axis_name="s")
    out_type = [
        jax.ShapeDtypeStruct((NPAD, 16), F32),
        jax.ShapeDtypeStruct((NSUB, NPAD, 16), F32),
    ]

    @functools.partial(
        pl.kernel,
        out_type=out_type,
        mesh=mesh,
        scratch_types=[
            pltpu.VMEM((CH,), jnp.int32),
            pltpu.VMEM((CH,), jnp.int32),
            pltpu.VMEM((CH, 16), F32),
            pltpu.VMEM((HALFPAD, 16), F32),
            pltpu.VMEM_SHARED((NPAD, 16), F32),
        ],
        compiler_params=_sc_compiler_params(),
    )
    def k(pb_hbm, src_hbm, dst_hbm, osum, omaxp, idxs, idxd, rows, mac, ssum):
        c = lax.axis_index("c")
        s = lax.axis_index("s")
        base = c * HALF

        neg = jnp.full((16,), NEG, F32)

        @pl.loop(0, HALFPAD)
        def _(i):
            mac[i, :] = neg

        zero = jnp.zeros((16,), F32)

        @pl.loop(0, 626)
        def _(i):
            rows[i, :] = zero

        pltpu.sync_copy(rows.at[pl.ds(0, 626)], ssum.at[pl.ds(s * 626, 626)])
        plsc.subcore_barrier()

        iot = lax.iota(jnp.int32, 16)

        @pl.loop(0, NCHUNK)
        def _(ci):
            e0 = s * EDGES_PER_SLOT + ci * CH
            pltpu.sync_copy(src_hbm.at[pl.ds(e0, CH)], idxs)
            pltpu.sync_copy(dst_hbm.at[pl.ds(e0, CH)], idxd)
            pltpu.sync_copy(pb_hbm.at[idxs], rows)

            @pl.when(c == 0)
            def _():
                pltpu.sync_copy(rows, ssum.at[idxd], add=True)

            @pl.loop(0, CH // 16)
            def _(v):
                d16 = idxd[pl.ds(v * 16, 16)]
                local = d16 - base
                act = (local >= 0) & (local < HALF)
                sl = jnp.where(act, local, 0)
                rix = iot + v * 16
                cols = [
                    plsc.load_gather(rows, [rix, jnp.full((16,), j, jnp.int32)])
                    for j in range(d)
                ]

                def cond(a):
                    return jnp.max(a.astype(jnp.int32)) > 0

                def body(a):
                    _, win = plsc.scan_count(sl, mask=a)
                    win = win & a
                    for j in range(d):
                        cj = jnp.full((16,), j, jnp.int32)
                        cur = plsc.load_gather(mac, [sl, cj], mask=win)
                        plsc.store_scatter(
                            mac, [sl, cj], jnp.maximum(cur, cols[j]), mask=win
                        )
                    return a & jnp.logical_not(win)

                lax.while_loop(cond, body, act)

        plsc.subcore_barrier()

        @pl.when(c == 0)
        def _():
            pltpu.sync_copy(ssum.at[pl.ds(s * 626, 626)],
                            osum.at[pl.ds(s * 626, 626)])

        pltpu.sync_copy(mac, omaxp.at[s].at[pl.ds(c * HALFPAD, HALFPAD)])

    return k


def _bn_lrelu(h, g, be, rowmask, nvalid):
    hm = jnp.where(rowmask, h, 0.0)
    mu = jnp.sum(hm, axis=0, keepdims=True) / nvalid
    var = jnp.sum(jnp.where(rowmask, (h - mu) ** 2, 0.0), axis=0,
                  keepdims=True) / nvalid
    hn = (h - mu) * lax.rsqrt(var + 1e-5) * g + be
    hn = jnp.where(hn > 0, hn, 0.01 * hn)
    return jnp.where(rowmask, hn, 0.0)


def _combine(pa, bvec, osum, omaxp, d):
    cnt = osum[:, d:d + 1]
    pab = pa[:, :d] + bvec[:, :d]
    ssum = osum[:, :d] + cnt * pab
    smean = ssum / jnp.maximum(cnt, 1.0)
    m = jnp.max(omaxp, axis=0)
    mm = jnp.concatenate(
        [m[:HALF], m[HALFPAD:HALFPAD + HALF],
         jnp.zeros((NPAD - 2 * HALF, 16), F32)], axis=0)
    smax = jnp.where(cnt > 0, pab + mm[:, :d], 0.0)
    return jnp.concatenate([ssum, smean, smax], axis=1)


def _make_tc0():
    def body(x_ref, wcat_ref, pa_ref, pb_ref):
        h = jnp.dot(x_ref[...], wcat_ref[...], preferred_element_type=F32,
                    precision=lax.Precision.HIGHEST)
        pa_ref[...] = h[:, :16]
        lane = lax.broadcasted_iota(jnp.int32, (NPAD, 16), 1)
        pb_ref[...] = jnp.where(lane == 12, 1.0, h[:, 16:])

    return pl.pallas_call(
        body,
        out_shape=[jax.ShapeDtypeStruct((NPAD, 16), F32)] * 2,
    )


def _make_tcmid(d, d2):
    def body(pa_ref, bvec_ref, osum_ref, omaxp_ref, g_ref, be_ref,
             wcat_ref, pa2_ref, pb2_ref):
        rowmask = lax.broadcasted_iota(jnp.int32, (NPAD, 1), 0) < N
        h = _combine(pa_ref[...], bvec_ref[...], osum_ref[...],
                     omaxp_ref[...], d)
        h = _bn_lrelu(h, g_ref[...], be_ref[...], rowmask, float(N))
        nxt = jnp.dot(h, wcat_ref[...], preferred_element_type=F32,
                      precision=lax.Precision.HIGHEST)
        pa2_ref[...] = nxt[:, :16]
        lane = lax.broadcasted_iota(jnp.int32, (NPAD, 16), 1)
        pb2_ref[...] = jnp.where(lane == d2, 1.0, nxt[:, 16:])

    return pl.pallas_call(
        body,
        out_shape=[jax.ShapeDtypeStruct((NPAD, 16), F32)] * 2,
    )


def _make_tcfinal(d):
    def body(pa_ref, bvec_ref, osum_ref, omaxp_ref, g_ref, be_ref,
             l1w_ref, l1b_ref, l2w_ref, l2b_ref, g4_ref, be4_ref,
             ow_ref, ob_ref, out_ref):
        rowmask = lax.broadcasted_iota(jnp.int32, (NPAD, 1), 0) < N
        h = _combine(pa_ref[...], bvec_ref[...], osum_ref[...],
                     omaxp_ref[...], d)
        h = _bn_lrelu(h, g_ref[...], be_ref[...], rowmask, float(N))
        v = jnp.dot(h, l1w_ref[...], preferred_element_type=F32,
                    precision=lax.Precision.HIGHEST) + l1b_ref[...]
        v = jnp.where(v > 0, v, 0.01 * v)
        u = jnp.dot(v, l2w_ref[...], preferred_element_type=F32,
                    precision=lax.Precision.HIGHEST) + l2b_ref[...]
        u = _bn_lrelu(u, g4_ref[...], be4_ref[...], rowmask, float(N))
        o = jnp.dot(u, ow_ref[...], preferred_element_type=F32,
                    precision=lax.Precision.HIGHEST) + ob_ref[...]
        out_ref[...] = jnp.concatenate([o, jnp.zeros((NPAD, 7), F32)], axis=1)

    return pl.pallas_call(
        body,
        out_shape=jax.ShapeDtypeStruct((NPAD, 8), F32),
    )


def _pad_cols(a, w):
    return jnp.pad(a, ((0, 0), (0, w - a.shape[1])))


def _wcat(W, din):
    return jnp.concatenate(
        [_pad_cols(W[:, :din].T, 16), _pad_cols(W[:, din:].T, 16)], axis=1)


def _sc_edge(pb_pad, src, dst, d):
    return _make_sc_edge(d)(pb_pad, src, dst)


def kernel(x, edge_index, W1, b1, g1, be1, W2, b2, g2, be2, W3, b3, g3, be3,
           L1w, L1b, L2w, L2b, g4, be4, Ow, Ob):
    src = edge_index[0]
    dst = edge_index[1]
    x_pad = jnp.pad(x, ((0, NPAD - N), (0, 0)))

    row = lambda a: a.reshape(1, -1)

    pa, pb = _make_tc0()(x_pad, _wcat(W1, D))
    osum, omaxp = _sc_edge(pb, src, dst, 12)
    pa, pb = _make_tcmid(12, 9)(
        pa, row(_pad_cols(b1.reshape(1, -1), 16)), osum, omaxp,
        row(g1), row(be1), _wcat(W2, 36))
    osum, omaxp = _sc_edge(pb, src, dst, 9)
    pa, pb = _make_tcmid(9, 7)(
        pa, row(_pad_cols(b2.reshape(1, -1), 16)), osum, omaxp,
        row(g2), row(be2), _wcat(W3, 27))
    osum, omaxp = _sc_edge(pb, src, dst, 7)
    out = _make_tcfinal(7)(
        pa, row(_pad_cols(b3.reshape(1, -1), 16)), osum, omaxp,
        row(g3), row(be3),
        L1w.T, row(L1b), L2w.T, row(L2b), row(g4), row(be4),
        Ow.T, row(Ob))
    return out[:N, :1]


# speedup vs baseline: 9.3654x; 9.3654x over previous
import dataclasses
import functools

import jax
import jax.numpy as jnp
from jax import lax
from jax.experimental import pallas as pl
from jax.experimental.pallas import tpu as pltpu
from jax.experimental.pallas import tpu_sc as plsc

N = 10000
E = 320000
D = 128
NPAD = 10240
HALF = 5000
HALFPAD = 5120
NSUB = 16
CH = 2000
EDGES_PER_SLOT = E // NSUB
NCHUNK = EDGES_PER_SLOT // CH
NEG = -3e38
F32 = jnp.float32


def _sc_compiler_params():
    cp = pltpu.CompilerParams()
    fields = pltpu.CompilerParams.__dataclass_fields__
    if "needs_layout_passes" in fields:
        cp = dataclasses.replace(cp, needs_layout_passes=False)
    if "use_tc_tiling_on_sc" in fields:
        cp = dataclasses.replace(cp, use_tc_tiling_on_sc=False)
    return cp


def _make_sc_edge(d):
    mesh = plsc.VectorSubcoreMesh(core_axis_name="c", subcore_axis_name="s")
    out_type = [
        jax.ShapeDtypeStruct((NPAD, 16), F32),
        jax.ShapeDtypeStruct((NPAD, NSUB * 16), F32),
    ]

    @functools.partial(
        pl.kernel,
        out_type=out_type,
        mesh=mesh,
        scratch_types=[
            pltpu.VMEM((CH,), jnp.int32),
            pltpu.VMEM((CH,), jnp.int32),
            pltpu.VMEM((CH, 16), F32),
            pltpu.VMEM((HALFPAD, 16), F32),
            pltpu.VMEM((16,), jnp.int32),
            pltpu.VMEM((16,), jnp.int32),
            pltpu.VMEM_SHARED((NPAD, 16), F32),
        ],
        compiler_params=_sc_compiler_params(),
    )
    def k(pb_hbm, src_hbm, dst_hbm, osum, omaxp, idxs, idxd, rows, mac,
          kbuf, mbuf, ssum):
        c = lax.axis_index("c")
        s = lax.axis_index("s")
        base = c * HALF

        neg = jnp.full((16,), NEG, F32)

        @pl.loop(0, HALFPAD)
        def _(i):
            mac[i, :] = neg

        zero = jnp.zeros((16,), F32)

        @pl.loop(0, 640)
        def _(i):
            rows[i, :] = zero

        pltpu.sync_copy(rows.at[pl.ds(0, 640)], ssum.at[pl.ds(s * 640, 640)])
        plsc.subcore_barrier()

        iot = lax.iota(jnp.int32, 16)

        @pl.loop(0, NCHUNK)
        def _(ci):
            e0 = s * EDGES_PER_SLOT + ci * CH
            pltpu.sync_copy(src_hbm.at[pl.ds(e0, CH)], idxs)
            pltpu.sync_copy(dst_hbm.at[pl.ds(e0, CH)], idxd)
            pltpu.sync_copy(pb_hbm.at[idxs], rows)

            @pl.when(c == 0)
            def _():
                pltpu.sync_copy(rows, ssum.at[idxd], add=True)

            @pl.loop(0, CH // 16)
            def _(v):
                d16 = idxd[pl.ds(v * 16, 16)]
                local = d16 - base
                act = (local >= 0) & (local < HALF)
                sl = jnp.where(act, local, 0)
                rix = iot + v * 16
                cols = [
                    plsc.load_gather(rows, [rix, jnp.full((16,), j, jnp.int32)])
                    for j in range(d)
                ]

                def cond(carry):
                    a, r = carry
                    return (jnp.max(a.astype(jnp.int32)) > 0) & (r < 16)

                def body(carry):
                    a, r = carry
                    sk, sv = plsc.sort_key_val(sl, iot, mask=a)[:2]
                    nact = jnp.sum(a.astype(jnp.int32))
                    kbuf[...] = sk
                    prevk = plsc.load_gather(kbuf, [jnp.maximum(iot - 1, 0)])
                    first = (iot < nact) & ((iot == 0) | (sk != prevk))
                    plsc.store_scatter(mbuf, [sv], first.astype(jnp.int32))
                    win = a & (plsc.load_gather(mbuf, [iot]) > 0)
                    for j in range(d):
                        cj = jnp.full((16,), j, jnp.int32)
                        cur = plsc.load_gather(mac, [sl, cj], mask=win)
                        plsc.store_scatter(
                            mac, [sl, cj], jnp.maximum(cur, cols[j]), mask=win
                        )
                    return a & jnp.logical_not(win), r + 1

                lax.while_loop(cond, body, (act, jnp.int32(0)))

        plsc.subcore_barrier()

        @pl.when(c == 0)
        def _():
            pltpu.sync_copy(ssum.at[pl.ds(s * 640, 640)],
                            osum.at[pl.ds(s * 640, 640)])

        pltpu.sync_copy(
            mac, omaxp.at[pl.ds(c * HALFPAD, HALFPAD), pl.ds(s * 16, 16)])

    return k


def _bn_lrelu(h, g, be, rowmask, nvalid):
    hm = jnp.where(rowmask, h, 0.0)
    mu = jnp.sum(hm, axis=0, keepdims=True) / nvalid
    var = jnp.sum(jnp.where(rowmask, (h - mu) ** 2, 0.0), axis=0,
                  keepdims=True) / nvalid
    hn = (h - mu) * lax.rsqrt(var + 1e-5) * g + be
    hn = jnp.where(hn > 0, hn, 0.01 * hn)
    return jnp.where(rowmask, hn, 0.0)


def _make_maxcmb():
    RB = 1024

    def body(omaxp_ref, m_ref):
        m = omaxp_ref[:, :16]
        for s_ in range(1, NSUB):
            m = jnp.maximum(m, omaxp_ref[:, s_ * 16:(s_ + 1) * 16])
        m_ref[...] = m

    return pl.pallas_call(
        body,
        grid=(NPAD // RB,),
        in_specs=[pl.BlockSpec((RB, NSUB * 16), lambda i: (i, 0))],
        out_specs=pl.BlockSpec((RB, 16), lambda i: (i, 0)),
        out_shape=jax.ShapeDtypeStruct((NPAD, 16), F32),
    )


def _combine(packed, bvec, d):
    pa, osum, mm = packed[:, :16], packed[:, 16:32], packed[:, 32:48]
    cnt = osum[:, d:d + 1]
    pab = pa[:, :d] + bvec[:, :d]
    ssum = osum[:, :d] + cnt * pab
    smean = ssum / jnp.maximum(cnt, 1.0)
    smax = jnp.where(cnt > 0, pab + mm[:, :d], 0.0)
    return jnp.concatenate([ssum, smean, smax], axis=1)


def _make_tc0():
    def body(x_ref, wcat_ref, out_ref):
        h = jnp.dot(x_ref[...], wcat_ref[...], preferred_element_type=F32)
        lane = lax.broadcasted_iota(jnp.int32, (NPAD, 32), 1)
        out_ref[...] = jnp.where(lane == 16 + 12, 1.0, h)

    return pl.pallas_call(
        body,
        out_shape=jax.ShapeDtypeStruct((NPAD, 32), F32),
    )


def _make_tcmid(d, d2):
    def body(packed_ref, bvec_ref, g_ref, be_ref, wcat_ref, out_ref):
        rowmask = lax.broadcasted_iota(jnp.int32, (NPAD, 1), 0) < N
        h = _combine(packed_ref[...], bvec_ref[...], d)
        h = _bn_lrelu(h, g_ref[...], be_ref[...], rowmask, float(N))
        nxt = jnp.dot(h, wcat_ref[...], preferred_element_type=F32)
        lane = lax.broadcasted_iota(jnp.int32, (NPAD, 32), 1)
        out_ref[...] = jnp.where(lane == 16 + d2, 1.0, nxt)

    return pl.pallas_call(
        body,
        out_shape=jax.ShapeDtypeStruct((NPAD, 32), F32),
    )


def _make_tcfinal(d):
    def body(packed_ref, bvec_ref, g_ref, be_ref,
             l1w_ref, l1b_ref, l2w_ref, l2b_ref, g4_ref, be4_ref,
             ow_ref, ob_ref, out_ref):
        rowmask = lax.broadcasted_iota(jnp.int32, (NPAD, 1), 0) < N
        h = _combine(packed_ref[...], bvec_ref[...], d)
        h = _bn_lrelu(h, g_ref[...], be_ref[...], rowmask, float(N))
        v = jnp.dot(h, l1w_ref[...], preferred_element_type=F32) + l1b_ref[...]
        v = jnp.where(v > 0, v, 0.01 * v)
        u = jnp.dot(v, l2w_ref[...], preferred_element_type=F32) + l2b_ref[...]
        u = _bn_lrelu(u, g4_ref[...], be4_ref[...], rowmask, float(N))
        o = jnp.dot(u, ow_ref[...], preferred_element_type=F32) + ob_ref[...]
        out_ref[...] = jnp.concatenate([o, jnp.zeros((NPAD, 7), F32)], axis=1)

    return pl.pallas_call(
        body,
        out_shape=jax.ShapeDtypeStruct((NPAD, 8), F32),
    )


def _pad_cols(a, w):
    return jnp.pad(a, ((0, 0), (0, w - a.shape[1])))


def _wcat(W, din):
    return jnp.concatenate(
        [_pad_cols(W[:, :din].T, 16), _pad_cols(W[:, din:].T, 16)], axis=1)


def _sc_edge(pb_pad, src, dst, d):
    return _make_sc_edge(d)(pb_pad, src, dst)


def kernel(x, edge_index, W1, b1, g1, be1, W2, b2, g2, be2, W3, b3, g3, be3,
           L1w, L1b, L2w, L2b, g4, be4, Ow, Ob):
    src = edge_index[0]
    dst = edge_index[1]
    x_pad = jnp.pad(x, ((0, NPAD - N), (0, 0)))

    row = lambda a: a.reshape(1, -1)

    def pack(papb, osum, omaxp):
        mraw = _make_maxcmb()(omaxp)
        mm = jnp.concatenate(
            [mraw[:HALF], mraw[HALFPAD:HALFPAD + HALF],
             jnp.zeros((NPAD - 2 * HALF, 16), F32)], axis=0)
        return jnp.concatenate([papb[:, :16], osum, mm], axis=1)

    papb = _make_tc0()(x_pad, _wcat(W1, D))
    osum, omaxp = _sc_edge(papb[:, 16:], src, dst, 12)
    papb = _make_tcmid(12, 9)(
        pack(papb, osum, omaxp), row(_pad_cols(b1.reshape(1, -1), 16)),
        row(g1), row(be1), _wcat(W2, 36))
    osum, omaxp = _sc_edge(papb[:, 16:], src, dst, 9)
    papb = _make_tcmid(9, 7)(
        pack(papb, osum, omaxp), row(_pad_cols(b2.reshape(1, -1), 16)),
        row(g2), row(be2), _wcat(W3, 27))
    osum, omaxp = _sc_edge(papb[:, 16:], src, dst, 7)
    out = _make_tcfinal(7)(
        pack(papb, osum, omaxp), row(_pad_cols(b3.reshape(1, -1), 16)),
        row(g3), row(be3),
        L1w.T, row(L1b), L2w.T, row(L2b), row(g4), row(be4),
        Ow.T, row(Ob))
    return out[:N, :1]
